# baseline (speedup 1.0000x reference)
import functools

import jax
import jax.numpy as jnp
import numpy as np
from jax import lax
from jax.experimental import pallas as pl
from jax.experimental.pallas import tpu as pltpu
from jax.experimental.pallas import tpu_sc as plsc

N_ROWS = 16384
N_CLASSES_K = 1000
COLD = np.float32(0.1 / 1000.0)
HOT = np.float32(np.float32(1.0 - 0.1) + COLD)

NW = 32
ROWS_PER_W = N_ROWS // NW
CHUNK_ROWS = 32
NCHUNK = ROWS_PER_W // CHUNK_ROWS
NBUF = 2

_FILL_STARTS = list(range(0, N_CLASSES_K - 15, 16))
if _FILL_STARTS[-1] + 16 < N_CLASSES_K:
    _FILL_STARTS.append(N_CLASSES_K - 16)


def _poke(buf, tgt_v, c, value_vec):
    for j in range(CHUNK_ROWS // 16):
        tgt16 = tgt_v[pl.ds(c * CHUNK_ROWS + j * 16, 16)]
        row = lax.iota(jnp.int32, 16) + (j * 16)
        plsc.store_scatter(buf, [row, tgt16], value_vec)


def _sc_body(tgt_hbm, out_hbm, tgt_v, bufs, sems):
    wid = lax.axis_index("s") * 2 + lax.axis_index("c")
    base_row = wid * ROWS_PER_W

    pltpu.sync_copy(tgt_hbm.at[pl.ds(base_row, ROWS_PER_W)], tgt_v)

    cold_vec = jnp.full((16,), COLD, jnp.float32)
    hot_vec = jnp.full((16,), HOT, jnp.float32)

    def fill(i, _):
        for b in range(NBUF):
            for cs in _FILL_STARTS:
                bufs[b][i, pl.ds(cs, 16)] = cold_vec
        return 0

    lax.fori_loop(0, CHUNK_ROWS, fill, 0)

    copies = [None] * NCHUNK
    for c in range(NCHUNK):
        b = c % NBUF
        if c >= NBUF:
            copies[c - NBUF].wait()
            _poke(bufs[b], tgt_v, c - NBUF, cold_vec)
        _poke(bufs[b], tgt_v, c, hot_vec)
        copies[c] = pltpu.async_copy(
            bufs[b], out_hbm.at[pl.ds(base_row + c * CHUNK_ROWS, CHUNK_ROWS)],
            sems[b])
    for c in range(NCHUNK - NBUF, NCHUNK):
        copies[c].wait()


@functools.partial(
    pl.kernel,
    out_type=jax.ShapeDtypeStruct((N_ROWS, N_CLASSES_K), jnp.float32),
    mesh=plsc.VectorSubcoreMesh(
        core_axis_name="c", subcore_axis_name="s", num_cores=2, num_subcores=16),
    scratch_types=[
        pltpu.VMEM((ROWS_PER_W,), jnp.int32),
        [pltpu.VMEM((CHUNK_ROWS, N_CLASSES_K), jnp.float32) for _ in range(NBUF)],
        [pltpu.SemaphoreType.DMA for _ in range(NBUF)],
    ],
    compiler_params=pltpu.CompilerParams(needs_layout_passes=False, use_tc_tiling_on_sc=True),
)
def _sc_one_hot(tgt_hbm, out_hbm, tgt_v, bufs, sems):
    _sc_body(tgt_hbm, out_hbm, tgt_v, bufs, sems)


def kernel(target):
    return _sc_one_hot(target.astype(jnp.int32))


# speedup vs baseline: 1.2485x; 1.2485x over previous
import functools

import jax
import jax.numpy as jnp
import numpy as np
from jax import lax
from jax.experimental import pallas as pl
from jax.experimental.pallas import tpu as pltpu

N_ROWS = 16384
N_CLASSES_K = 1000
COLD = np.float32(0.1 / 1000.0)
HOT = np.float32(np.float32(1.0 - 0.1) + COLD)

CH = 512
NCH = N_ROWS // CH
NBUF = 4


def _body(tgt_ref, out_ref, *scratch):
    bufs = scratch[:NBUF]
    sems = scratch[NBUF:]
    col = lax.broadcasted_iota(jnp.int32, (CH, N_CLASSES_K), 1)
    copies = [None] * NCH
    for c in range(NCH):
        b = c % NBUF
        if c >= NBUF:
            copies[c - NBUF].wait()
        tgt = tgt_ref[c, 0, :].reshape(CH, 1)
        bufs[b][...] = jnp.where(col == tgt, HOT, COLD)
        copies[c] = pltpu.make_async_copy(
            bufs[b], out_ref.at[pl.ds(c * CH, CH)], sems[b])
        copies[c].start()
    for c in range(NCH - NBUF, NCH):
        copies[c].wait()


def kernel(target):
    tgt3 = target.astype(jnp.int32).reshape(NCH, 1, CH)
    out = pl.pallas_call(
        _body,
        grid=(1,),
        in_specs=[pl.BlockSpec((NCH, 1, CH), lambda i: (0, 0, 0))],
        out_specs=pl.BlockSpec(memory_space=pl.ANY),
        out_shape=jax.ShapeDtypeStruct((N_ROWS, N_CLASSES_K), jnp.float32),
        scratch_shapes=(
            [pltpu.VMEM((CH, N_CLASSES_K), jnp.float32) for _ in range(NBUF)]
            + [pltpu.SemaphoreType.DMA for _ in range(NBUF)]
        ),
    )(tgt3)
    return out


# speedup vs baseline: 4.2868x; 3.4334x over previous
import functools

import jax
import jax.numpy as jnp
import numpy as np
from jax import lax
from jax.experimental import pallas as pl

N_ROWS = 16384
N_CLASSES_K = 1000
COLD = np.float32(0.1 / 1000.0)
HOT = np.float32(np.float32(1.0 - 0.1) + COLD)

CB = 40
NB = N_CLASSES_K // CB


def _body(tgt_ref, out_ref):
    j = pl.program_id(0)
    cls = lax.broadcasted_iota(jnp.int32, (CB, N_ROWS), 0) + j * CB
    tgt = tgt_ref[...]
    out_ref[...] = jnp.where(tgt == cls, HOT, COLD)


def kernel(target):
    tgt2 = target.astype(jnp.int32).reshape(1, N_ROWS)
    out_t = pl.pallas_call(
        _body,
        grid=(NB,),
        in_specs=[pl.BlockSpec((1, N_ROWS), lambda j: (0, 0))],
        out_specs=pl.BlockSpec((CB, N_ROWS), lambda j: (j, 0)),
        out_shape=jax.ShapeDtypeStruct((N_CLASSES_K, N_ROWS), jnp.float32),
    )(tgt2)
    return out_t.T
